# baseline (speedup 1.0000x reference)
import functools
import math

import jax
import jax.numpy as jnp
from jax import lax
from jax.experimental import pallas as pl
from jax.experimental.pallas import tpu as pltpu
from jax.experimental.pallas import tpu_sc as plsc

_NUM_WORDS = 100000
_NUM_SAMPLES = 8192
_EMB_DIM = 128
_BATCH = 4096
_LOG_NW_P1 = math.log(_NUM_WORDS + 1)
_BB = 512
_CHUNK = 128
_SHIFT = 16.0
_LOG2E = 1.4426950408889634
_BM = 100352


def _sc_gather(W, b, sampled_ids, targets, zeros_bm, ones_sc):
    D = W.shape[1]
    info = plsc.get_sparse_core_info()
    nc = info.num_cores
    nw = nc * info.num_subcores
    s_per = _NUM_SAMPLES // nw
    t_per = _BATCH // nw
    sc_per = _NUM_SAMPLES // nw
    z_per = _BM // info.num_subcores
    mesh = plsc.VectorSubcoreMesh(core_axis_name="c", subcore_axis_name="s")

    @functools.partial(
        pl.kernel,
        mesh=mesh,
        out_type=[
            jax.ShapeDtypeStruct((_NUM_SAMPLES + _BATCH, D), jnp.float32),
            jax.ShapeDtypeStruct((_NUM_SAMPLES,), jnp.float32),
            jax.ShapeDtypeStruct((_BATCH,), jnp.float32),
            jax.ShapeDtypeStruct((nc * _BM,), jnp.float32),
            jax.ShapeDtypeStruct((nc * _BATCH,), jnp.float32),
        ],
        scratch_types=[
            pltpu.VMEM((s_per,), jnp.int32),
            pltpu.VMEM((t_per,), jnp.int32),
            pltpu.VMEM((sc_per // _CHUNK, _CHUNK), jnp.int32),
            pltpu.VMEM((t_per * 2,), jnp.int32),
            pltpu.VMEM((s_per, D), jnp.float32),
            pltpu.VMEM((t_per, D), jnp.float32),
            pltpu.VMEM((s_per,), jnp.float32),
            pltpu.VMEM((t_per,), jnp.float32),
            pltpu.VMEM((z_per,), jnp.float32),
            pltpu.VMEM((_CHUNK,), jnp.float32),
            pltpu.VMEM((t_per * 2,), jnp.float32),
            pltpu.SemaphoreType.DMA,
            pltpu.SemaphoreType.DMA,
        ],
    )
    def gather_kernel(w_hbm, b_hbm, sid_hbm, tid_hbm, zz_hbm, oo_hbm,
                      rows_out, sbias_out, tbias_out, bm_out, pres_out,
                      sidx_v, tidx_v, scidx_v, pidx_v,
                      srows_v, trows_v, sbias_v, tbias_v,
                      zeros_v, ones_v, pres_v,
                      sem_r, sem_b):
        cid = lax.axis_index("c")
        sid_ax = lax.axis_index("s")
        wid = sid_ax * nc + cid
        s_base = wid * s_per
        t_base = wid * t_per

        pltpu.sync_copy(sid_hbm.at[pl.ds(s_base, s_per)], sidx_v)
        pltpu.sync_copy(tid_hbm.at[pl.ds(t_base, t_per)], tidx_v)
        copies = []
        for j in range(s_per // _CHUNK):
            sl = pl.ds(j * _CHUNK, _CHUNK)
            copies.append(
                pltpu.async_copy(w_hbm.at[sidx_v.at[sl]], srows_v.at[sl],
                                 sem_r))
            copies.append(
                pltpu.async_copy(b_hbm.at[sidx_v.at[sl]], sbias_v.at[sl],
                                 sem_b))
        copies.append(pltpu.async_copy(w_hbm.at[tidx_v], trows_v, sem_r))
        copies.append(pltpu.async_copy(b_hbm.at[tidx_v], tbias_v, sem_b))

        pltpu.sync_copy(zz_hbm, zeros_v)
        pltpu.sync_copy(zeros_v,
                        bm_out.at[pl.ds(cid * _BM + sid_ax * z_per, z_per)])

        for c in copies:
            c.wait()
        pltpu.sync_copy(srows_v, rows_out.at[pl.ds(s_base, s_per)])
        pltpu.sync_copy(trows_v,
                        rows_out.at[pl.ds(_NUM_SAMPLES + t_base, t_per)])
        pltpu.sync_copy(sbias_v, sbias_out.at[pl.ds(s_base, s_per)])
        pltpu.sync_copy(tbias_v, tbias_out.at[pl.ds(t_base, t_per)])

        plsc.subcore_barrier()

        sc_base = (cid * info.num_subcores + sid_ax) * sc_per
        for j in range(sc_per // _CHUNK):
            pltpu.sync_copy(sid_hbm.at[pl.ds(sc_base + j * _CHUNK, _CHUNK)],
                            scidx_v.at[j])
        pltpu.sync_copy(oo_hbm, ones_v)
        base_off = cid * _BM
        for j in range(sc_per // _CHUNK):
            for k in range(_CHUNK // 16):
                kk = pl.ds(k * 16, 16)
                scidx_v[j, kk] = scidx_v[j, kk] + base_off
        scat = []
        for j in range(sc_per // _CHUNK):
            scat.append(
                pltpu.async_copy(ones_v, bm_out.at[scidx_v.at[j]], sem_r))
        for c in scat:
            c.wait()

        plsc.subcore_barrier()

        pltpu.sync_copy(tid_hbm.at[pl.ds(sid_ax * (t_per * 2), t_per * 2)],
                        pidx_v)
        for k in range(t_per * 2 // 16):
            kk = pl.ds(k * 16, 16)
            pidx_v[kk] = pidx_v[kk] + base_off
        pg = []
        for j in range(t_per * 2 // _CHUNK):
            sl = pl.ds(j * _CHUNK, _CHUNK)
            pg.append(
                pltpu.async_copy(bm_out.at[pidx_v.at[sl]], pres_v.at[sl],
                                 sem_b))
        for c in pg:
            c.wait()
        pltpu.sync_copy(pres_v,
                        pres_out.at[pl.ds(cid * _BATCH + sid_ax * (t_per * 2),
                                          t_per * 2)])

    return gather_kernel(W, b, sampled_ids, targets, zeros_bm, ones_sc)


def _tc_body(nt_ref, emb_ref, sw_ref, tw_ref, sb_ref, tb_ref,
             tid_ref, sid_ref, p0_ref, p1_ref, out_ref, corr_ref, swb_ref):
    i = pl.program_id(0)
    nt = nt_ref[0, 0]
    inv_log = 1.0 / _LOG_NW_P1

    @pl.when(i == 0)
    def _():
        swb_ref[...] = sw_ref[...].astype(jnp.bfloat16)
        sf0 = sid_ref[...].astype(jnp.float32)
        sp = jnp.log((sf0 + 2.0) / (sf0 + 1.0)) * inv_log
        s_exp = 1.0 - jnp.exp(nt * jnp.log1p(-sp))
        corr_ref[...] = (sb_ref[...] - jnp.log(s_exp + 1e-7)
                         - _SHIFT) * _LOG2E

    emb = emb_ref[...]
    embs = (emb * _LOG2E).astype(jnp.bfloat16)
    y = lax.dot_general(
        embs, swb_ref[...],
        (((1,), (1,)), ((), ())),
        preferred_element_type=jnp.float32)
    y = y + corr_ref[...]

    s_col = jnp.sum(jnp.exp2(y), axis=1, keepdims=True)
    s_row = jnp.transpose(s_col)

    tdot_col = jnp.sum(tw_ref[...] * emb, axis=1, keepdims=True)
    tdot_row = jnp.transpose(tdot_col)

    tf = tid_ref[...].astype(jnp.float32)
    tp = jnp.log((tf + 2.0) / (tf + 1.0)) * inv_log
    t_exp = 1.0 - jnp.exp(nt * jnp.log1p(-tp))
    tl = tdot_row + tb_ref[...] - jnp.log(t_exp + 1e-7)

    absent = (1.0 - p0_ref[...]) * (1.0 - p1_ref[...])
    s_tot = s_row + absent * jnp.exp2((tl - _SHIFT) * _LOG2E)
    part = jnp.sum(_SHIFT + jnp.log(s_tot) - tl)

    @pl.when(i == 0)
    def _():
        out_ref[0, 0] = part

    @pl.when(i != 0)
    def _():
        out_ref[0, 0] += part


def _fused_loss(nt, embeddings, rows, sbias, tbias, tid2, sid2, p0, p1):
    nb = _BATCH // _BB
    return pl.pallas_call(
        _tc_body,
        grid=(nb,),
        in_specs=[
            pl.BlockSpec(memory_space=pltpu.SMEM),
            pl.BlockSpec((_BB, _EMB_DIM), lambda i: (i, 0)),
            pl.BlockSpec((_NUM_SAMPLES, _EMB_DIM), lambda i: (0, 0)),
            pl.BlockSpec((_BB, _EMB_DIM),
                         lambda i: (_NUM_SAMPLES // _BB + i, 0)),
            pl.BlockSpec((1, _NUM_SAMPLES), lambda i: (0, 0)),
            pl.BlockSpec((1, _BB), lambda i: (0, i)),
            pl.BlockSpec((1, _BB), lambda i: (0, i)),
            pl.BlockSpec((1, _NUM_SAMPLES), lambda i: (0, 0)),
            pl.BlockSpec((1, _BB), lambda i: (0, i)),
            pl.BlockSpec((1, _BB), lambda i: (0, i)),
        ],
        out_specs=pl.BlockSpec(memory_space=pltpu.SMEM),
        out_shape=jax.ShapeDtypeStruct((1, 1), jnp.float32),
        scratch_shapes=[
            pltpu.VMEM((1, _NUM_SAMPLES), jnp.float32),
            pltpu.VMEM((_NUM_SAMPLES, _EMB_DIM), jnp.bfloat16),
        ],
        compiler_params=pltpu.CompilerParams(
            dimension_semantics=("arbitrary",)),
    )(nt, embeddings, rows, rows, sbias, tbias, tid2, sid2, p0, p1)


def kernel(embeddings, targets, W, b, sampled_ids, num_tries):
    zeros_bm = jnp.zeros((_BM // 16,), jnp.float32)
    ones_sc = jnp.ones((_CHUNK,), jnp.float32)
    rows, sbias, tbias, _, pres = _sc_gather(
        W, b, sampled_ids, targets, zeros_bm, ones_sc)
    sb = sbias.reshape(1, _NUM_SAMPLES)
    tb = tbias.reshape(1, _BATCH)
    tid2 = targets.reshape(1, _BATCH)
    sid2 = sampled_ids.reshape(1, _NUM_SAMPLES)
    p0 = pres[:_BATCH].reshape(1, _BATCH)
    p1 = pres[_BATCH:].reshape(1, _BATCH)
    nt = jnp.asarray(num_tries, jnp.float32).reshape(1, 1)
    loss = _fused_loss(nt, embeddings, rows, sb, tb, tid2, sid2, p0, p1)
    return loss[0, 0]


# speedup vs baseline: 1.3988x; 1.3988x over previous
import functools
import math

import jax
import jax.numpy as jnp
from jax import lax
from jax.experimental import pallas as pl
from jax.experimental.pallas import tpu as pltpu
from jax.experimental.pallas import tpu_sc as plsc

_NUM_WORDS = 100000
_NUM_SAMPLES = 8192
_EMB_DIM = 128
_BATCH = 4096
_LOG_NW_P1 = math.log(_NUM_WORDS + 1)
_BB = 512
_CHUNK = 128
_SHIFT = 16.0
_LOG2E = 1.4426950408889634


def _sc_gather(W, b, sampled_ids, targets):
    D = W.shape[1]
    info = plsc.get_sparse_core_info()
    nw = info.num_cores * info.num_subcores
    s_per = _NUM_SAMPLES // nw
    t_per = _BATCH // nw
    mesh = plsc.VectorSubcoreMesh(core_axis_name="c", subcore_axis_name="s")

    @functools.partial(
        pl.kernel,
        mesh=mesh,
        out_type=[
            jax.ShapeDtypeStruct((_NUM_SAMPLES + _BATCH, D), jnp.float32),
            jax.ShapeDtypeStruct((_NUM_SAMPLES,), jnp.float32),
            jax.ShapeDtypeStruct((_BATCH,), jnp.float32),
        ],
        scratch_types=[
            pltpu.VMEM((s_per,), jnp.int32),
            pltpu.VMEM((t_per,), jnp.int32),
            pltpu.VMEM((s_per, D), jnp.float32),
            pltpu.VMEM((t_per, D), jnp.float32),
            pltpu.VMEM((s_per,), jnp.float32),
            pltpu.VMEM((t_per,), jnp.float32),
            pltpu.SemaphoreType.DMA,
            pltpu.SemaphoreType.DMA,
        ],
    )
    def gather_kernel(w_hbm, b_hbm, sid_hbm, tid_hbm,
                      rows_out, sbias_out, tbias_out,
                      sidx_v, tidx_v, srows_v, trows_v, sbias_v, tbias_v,
                      sem_r, sem_b):
        wid = lax.axis_index("s") * info.num_cores + lax.axis_index("c")
        s_base = wid * s_per
        t_base = wid * t_per
        pltpu.sync_copy(sid_hbm.at[pl.ds(s_base, s_per)], sidx_v)
        pltpu.sync_copy(tid_hbm.at[pl.ds(t_base, t_per)], tidx_v)
        copies = []
        for j in range(s_per // _CHUNK):
            sl = pl.ds(j * _CHUNK, _CHUNK)
            copies.append(
                pltpu.async_copy(w_hbm.at[sidx_v.at[sl]], srows_v.at[sl],
                                 sem_r))
            copies.append(
                pltpu.async_copy(b_hbm.at[sidx_v.at[sl]], sbias_v.at[sl],
                                 sem_b))
        copies.append(pltpu.async_copy(w_hbm.at[tidx_v], trows_v, sem_r))
        copies.append(pltpu.async_copy(b_hbm.at[tidx_v], tbias_v, sem_b))
        for c in copies:
            c.wait()
        pltpu.sync_copy(srows_v, rows_out.at[pl.ds(s_base, s_per)])
        pltpu.sync_copy(trows_v,
                        rows_out.at[pl.ds(_NUM_SAMPLES + t_base, t_per)])
        pltpu.sync_copy(sbias_v, sbias_out.at[pl.ds(s_base, s_per)])
        pltpu.sync_copy(tbias_v, tbias_out.at[pl.ds(t_base, t_per)])

    return gather_kernel(W, b, sampled_ids, targets)


def _tc_body(nt_ref, emb_ref, sw_ref, tw_ref, sb_ref, tb_ref,
             tid_ref, sid_ref, out_ref, corr_ref, swb_ref):
    i = pl.program_id(0)
    nt = nt_ref[0, 0]
    inv_log = 1.0 / _LOG_NW_P1

    @pl.when(i == 0)
    def _():
        swb_ref[...] = sw_ref[...].astype(jnp.bfloat16)
        sf0 = sid_ref[...].astype(jnp.float32)
        sp = jnp.log((sf0 + 2.0) / (sf0 + 1.0)) * inv_log
        s_exp = 1.0 - jnp.exp(nt * jnp.log1p(-sp))
        corr_ref[...] = (sb_ref[...] - jnp.log(s_exp + 1e-7)
                         - _SHIFT) * _LOG2E

    emb = emb_ref[...]
    embs = (emb * _LOG2E).astype(jnp.bfloat16)
    y = lax.dot_general(
        embs, swb_ref[...],
        (((1,), (1,)), ((), ())),
        preferred_element_type=jnp.float32)
    y = y + corr_ref[...]

    tid_row = tid_ref[...]
    tid_col = jnp.transpose(tid_row)
    y = jnp.where(sid_ref[...] == tid_col, -20000.0, y)

    s_col = jnp.sum(jnp.exp2(y), axis=1, keepdims=True)
    s_row = jnp.transpose(s_col)

    tdot_col = jnp.sum(tw_ref[...] * emb, axis=1, keepdims=True)
    tdot_row = jnp.transpose(tdot_col)

    tf = tid_row.astype(jnp.float32)
    tp = jnp.log((tf + 2.0) / (tf + 1.0)) * inv_log
    t_exp = 1.0 - jnp.exp(nt * jnp.log1p(-tp))
    tl = tdot_row + tb_ref[...] - jnp.log(t_exp + 1e-7)

    s_tot = s_row + jnp.exp2((tl - _SHIFT) * _LOG2E)
    part = jnp.sum(_SHIFT + jnp.log(s_tot) - tl)

    @pl.when(i == 0)
    def _():
        out_ref[0, 0] = part

    @pl.when(i != 0)
    def _():
        out_ref[0, 0] += part


def _fused_loss(nt, embeddings, rows, sbias, tbias, tid2, sid2):
    nb = _BATCH // _BB
    return pl.pallas_call(
        _tc_body,
        grid=(nb,),
        in_specs=[
            pl.BlockSpec(memory_space=pltpu.SMEM),
            pl.BlockSpec((_BB, _EMB_DIM), lambda i: (i, 0)),
            pl.BlockSpec((_NUM_SAMPLES, _EMB_DIM), lambda i: (0, 0)),
            pl.BlockSpec((_BB, _EMB_DIM),
                         lambda i: (_NUM_SAMPLES // _BB + i, 0)),
            pl.BlockSpec((1, _NUM_SAMPLES), lambda i: (0, 0)),
            pl.BlockSpec((1, _BB), lambda i: (0, i)),
            pl.BlockSpec((1, _BB), lambda i: (0, i)),
            pl.BlockSpec((1, _NUM_SAMPLES), lambda i: (0, 0)),
        ],
        out_specs=pl.BlockSpec(memory_space=pltpu.SMEM),
        out_shape=jax.ShapeDtypeStruct((1, 1), jnp.float32),
        scratch_shapes=[
            pltpu.VMEM((1, _NUM_SAMPLES), jnp.float32),
            pltpu.VMEM((_NUM_SAMPLES, _EMB_DIM), jnp.bfloat16),
        ],
        compiler_params=pltpu.CompilerParams(
            dimension_semantics=("arbitrary",)),
    )(nt, embeddings, rows, rows, sbias, tbias, tid2, sid2)


def kernel(embeddings, targets, W, b, sampled_ids, num_tries):
    rows, sbias, tbias = _sc_gather(W, b, sampled_ids, targets)
    sb = sbias.reshape(1, _NUM_SAMPLES)
    tb = tbias.reshape(1, _BATCH)
    tid2 = targets.reshape(1, _BATCH)
    sid2 = sampled_ids.reshape(1, _NUM_SAMPLES)
    nt = jnp.asarray(num_tries, jnp.float32).reshape(1, 1)
    loss = _fused_loss(nt, embeddings, rows, sb, tb, tid2, sid2)
    return loss[0, 0]


# speedup vs baseline: 1.4154x; 1.0118x over previous
import functools
import math

import jax
import jax.numpy as jnp
from jax import lax
from jax.experimental import pallas as pl
from jax.experimental.pallas import tpu as pltpu
from jax.experimental.pallas import tpu_sc as plsc

_NUM_WORDS = 100000
_NUM_SAMPLES = 8192
_EMB_DIM = 128
_BATCH = 4096
_LOG_NW_P1 = math.log(_NUM_WORDS + 1)
_BB = 512
_CHUNK = 128
_SHIFT = 16.0
_LOG2E = 1.4426950408889634


def _sc_gather(W, b, sampled_ids, targets):
    D = W.shape[1]
    info = plsc.get_sparse_core_info()
    nw = info.num_cores * info.num_subcores
    s_per = _NUM_SAMPLES // nw
    t_per = _BATCH // nw
    mesh = plsc.VectorSubcoreMesh(core_axis_name="c", subcore_axis_name="s")

    @functools.partial(
        pl.kernel,
        mesh=mesh,
        out_type=[
            jax.ShapeDtypeStruct((_NUM_SAMPLES + _BATCH, D), jnp.float32),
            jax.ShapeDtypeStruct((_NUM_SAMPLES,), jnp.float32),
            jax.ShapeDtypeStruct((_BATCH,), jnp.float32),
        ],
        scratch_types=[
            pltpu.VMEM((s_per,), jnp.int32),
            pltpu.VMEM((t_per,), jnp.int32),
            pltpu.VMEM((s_per, D), jnp.float32),
            pltpu.VMEM((t_per, D), jnp.float32),
            pltpu.VMEM((s_per,), jnp.float32),
            pltpu.VMEM((t_per,), jnp.float32),
            pltpu.SemaphoreType.DMA,
            pltpu.SemaphoreType.DMA,
        ],
    )
    def gather_kernel(w_hbm, b_hbm, sid_hbm, tid_hbm,
                      rows_out, sbias_out, tbias_out,
                      sidx_v, tidx_v, srows_v, trows_v, sbias_v, tbias_v,
                      sem_r, sem_b):
        wid = lax.axis_index("s") * info.num_cores + lax.axis_index("c")
        s_base = wid * s_per
        t_base = wid * t_per
        pltpu.sync_copy(sid_hbm.at[pl.ds(s_base, s_per)], sidx_v)
        pltpu.sync_copy(tid_hbm.at[pl.ds(t_base, t_per)], tidx_v)
        copies = []
        for j in range(s_per // _CHUNK):
            sl = pl.ds(j * _CHUNK, _CHUNK)
            copies.append(
                pltpu.async_copy(w_hbm.at[sidx_v.at[sl]], srows_v.at[sl],
                                 sem_r))
            copies.append(
                pltpu.async_copy(b_hbm.at[sidx_v.at[sl]], sbias_v.at[sl],
                                 sem_b))
        copies.append(pltpu.async_copy(w_hbm.at[tidx_v], trows_v, sem_r))
        copies.append(pltpu.async_copy(b_hbm.at[tidx_v], tbias_v, sem_b))
        for c in copies:
            c.wait()
        pltpu.sync_copy(srows_v, rows_out.at[pl.ds(s_base, s_per)])
        pltpu.sync_copy(trows_v,
                        rows_out.at[pl.ds(_NUM_SAMPLES + t_base, t_per)])
        pltpu.sync_copy(sbias_v, sbias_out.at[pl.ds(s_base, s_per)])
        pltpu.sync_copy(tbias_v, tbias_out.at[pl.ds(t_base, t_per)])

    return gather_kernel(W, b, sampled_ids, targets)


def _tc_body(nt_ref, emb_ref, sw_ref, tw_ref, sb_ref, tb_ref,
             tid_ref, sid_ref, out_ref, corr_ref, swb_ref):
    i = pl.program_id(0)
    nt = nt_ref[0, 0]
    inv_log = 1.0 / _LOG_NW_P1

    @pl.when(i == 0)
    def _():
        swb_ref[...] = sw_ref[...].astype(jnp.bfloat16)
        sf0 = sid_ref[...].astype(jnp.float32)
        sp = jnp.log((sf0 + 2.0) / (sf0 + 1.0)) * inv_log
        s_exp = 1.0 - jnp.exp(nt * jnp.log1p(-sp))
        corr_ref[...] = (sb_ref[...] - jnp.log(s_exp + 1e-7)
                         - _SHIFT) * _LOG2E

    emb = emb_ref[...]
    embs = (emb * _LOG2E).astype(jnp.bfloat16)
    tid_row = tid_ref[...]
    tid_col = jnp.transpose(tid_row)

    ns_sl = _NUM_SAMPLES // 4
    s_col = None
    for k in range(4):
        sl = pl.ds(k * ns_sl, ns_sl)
        yk = lax.dot_general(
            embs, swb_ref[sl, :],
            (((1,), (1,)), ((), ())),
            preferred_element_type=jnp.float32)
        yk = yk + corr_ref[:, sl]
        yk = jnp.where(sid_ref[:, sl] == tid_col, -20000.0, yk)
        sk = jnp.sum(jnp.exp2(yk), axis=1, keepdims=True)
        s_col = sk if s_col is None else s_col + sk
    s_row = jnp.transpose(s_col)

    tdot_col = jnp.sum(tw_ref[...] * emb, axis=1, keepdims=True)
    tdot_row = jnp.transpose(tdot_col)

    tf = tid_row.astype(jnp.float32)
    tp = jnp.log((tf + 2.0) / (tf + 1.0)) * inv_log
    t_exp = 1.0 - jnp.exp(nt * jnp.log1p(-tp))
    tl = tdot_row + tb_ref[...] - jnp.log(t_exp + 1e-7)

    s_tot = s_row + jnp.exp2((tl - _SHIFT) * _LOG2E)
    part = jnp.sum(_SHIFT + jnp.log(s_tot) - tl)

    @pl.when(i == 0)
    def _():
        out_ref[0, 0] = part

    @pl.when(i != 0)
    def _():
        out_ref[0, 0] += part


def _fused_loss(nt, embeddings, rows, sbias, tbias, tid2, sid2):
    nb = _BATCH // _BB
    return pl.pallas_call(
        _tc_body,
        grid=(nb,),
        in_specs=[
            pl.BlockSpec(memory_space=pltpu.SMEM),
            pl.BlockSpec((_BB, _EMB_DIM), lambda i: (i, 0)),
            pl.BlockSpec((_NUM_SAMPLES, _EMB_DIM), lambda i: (0, 0)),
            pl.BlockSpec((_BB, _EMB_DIM),
                         lambda i: (_NUM_SAMPLES // _BB + i, 0)),
            pl.BlockSpec((1, _NUM_SAMPLES), lambda i: (0, 0)),
            pl.BlockSpec((1, _BB), lambda i: (0, i)),
            pl.BlockSpec((1, _BB), lambda i: (0, i)),
            pl.BlockSpec((1, _NUM_SAMPLES), lambda i: (0, 0)),
        ],
        out_specs=pl.BlockSpec(memory_space=pltpu.SMEM),
        out_shape=jax.ShapeDtypeStruct((1, 1), jnp.float32),
        scratch_shapes=[
            pltpu.VMEM((1, _NUM_SAMPLES), jnp.float32),
            pltpu.VMEM((_NUM_SAMPLES, _EMB_DIM), jnp.bfloat16),
        ],
        compiler_params=pltpu.CompilerParams(
            dimension_semantics=("arbitrary",)),
    )(nt, embeddings, rows, rows, sbias, tbias, tid2, sid2)


def kernel(embeddings, targets, W, b, sampled_ids, num_tries):
    rows, sbias, tbias = _sc_gather(W, b, sampled_ids, targets)
    sb = sbias.reshape(1, _NUM_SAMPLES)
    tb = tbias.reshape(1, _BATCH)
    tid2 = targets.reshape(1, _BATCH)
    sid2 = sampled_ids.reshape(1, _NUM_SAMPLES)
    nt = jnp.asarray(num_tries, jnp.float32).reshape(1, 1)
    loss = _fused_loss(nt, embeddings, rows, sb, tb, tid2, sid2)
    return loss[0, 0]
